# baseline (speedup 1.0000x reference)
import functools

import jax
import jax.numpy as jnp
from jax import lax
from jax.experimental import pallas as pl
from jax.experimental.pallas import tpu as pltpu
from jax.experimental.pallas import tpu_sc as plsc

_EMBED = 64
_OUT = 2
_SEQ = 200
_BATCH = 4096
_SEQP = 256
_PAD = _SEQP - _SEQ
_G1 = 128
_G2 = _SEQ - _G1
_NC = 2
_NS = 16
_NW = _NC * _NS
_BPW = _BATCH // _NW


def _sc_body(textp_hbm, table_hbm, sums_hbm, idx_v, idx2_v, rows_v, out_v,
             sem):
    wid = lax.axis_index("s") * _NC + lax.axis_index("c")
    base = wid * _BPW
    pltpu.sync_copy(textp_hbm.at[pl.ds(base * 2, 2 * _BPW)], idx_v)

    def shift_row(r, carry):
        for k in range(8):
            sl = pl.ds(k * 16, 16)
            idx2_v[r, sl] = lax.shift_right_logical(idx_v[r, sl], 1)
        return carry

    lax.fori_loop(0, 2 * _BPW, shift_row, None, unroll=4)

    def one(b, carry):
        h1 = pltpu.async_copy(
            table_hbm.at[idx2_v.at[2 * b]],
            rows_v.at[pl.ds(0, _G1)], sem)
        h2 = pltpu.async_copy(
            table_hbm.at[idx2_v.at[2 * b + 1, pl.ds(0, _G2)]],
            rows_v.at[pl.ds(_G1, _G2)], sem)
        h1.wait()
        h2.wait()
        zero = jnp.zeros((16,), jnp.float32)

        def group(g, accs, row0_idx, par_row, nrows):
            par = idx_v[par_row, pl.ds(g * 16, 16)] & 1
            for j in range(nrows):
                off = par[j] * _EMBED
                s = row0_idx + g * 16 + j
                accs = tuple(
                    accs[c] + rows_v[s, pl.ds(off + c * 16, 16)]
                    for c in range(4))
            return accs

        accs = lax.fori_loop(
            0, _G1 // 16,
            lambda g, a: group(g, a, 0, 2 * b, 16),
            (zero,) * 4)
        accs = lax.fori_loop(
            0, _G2 // 16,
            lambda g, a: group(g, a, _G1, 2 * b + 1, 16),
            accs)
        accs = group(_G2 // 16, accs, _G1, 2 * b + 1, _G2 % 16)
        half = (b & 1) * _EMBED
        for c in range(4):
            out_v[b // 2, pl.ds(half + c * 16, 16)] = accs[c]
        return carry

    lax.fori_loop(0, _BPW, one, None)
    pltpu.sync_copy(out_v, sums_hbm.at[pl.ds(wid * (_BPW // 2), _BPW // 2)])


def _sc_gather_sum(textp, table2):
    f = pl.kernel(
        _sc_body,
        out_type=jax.ShapeDtypeStruct((_BATCH // 2, 2 * _EMBED), jnp.float32),
        mesh=plsc.VectorSubcoreMesh(core_axis_name="c", subcore_axis_name="s"),
        scratch_types=[
            pltpu.VMEM((2 * _BPW, 128), jnp.int32),
            pltpu.VMEM((2 * _BPW, 128), jnp.int32),
            pltpu.VMEM((_SEQP, 128), jnp.float32),
            pltpu.VMEM((_BPW // 2, 2 * _EMBED), jnp.float32),
            pltpu.SemaphoreType.DMA,
        ],
    )
    return f(textp, table2)


def _tc_body(sums_ref, text_ref, row0_ref, w_ref, b_ref, out_ref):
    cnt = jnp.sum((text_ref[...] == 0).astype(jnp.float32),
                  axis=1, keepdims=True) - float(_PAD)
    pooled = (sums_ref[...] - cnt * row0_ref[...]) * (1.0 / _SEQ)
    out_ref[...] = lax.dot_general(
        pooled, w_ref[...], (((1,), (1,)), ((), ())),
        preferred_element_type=jnp.float32,
        precision=lax.Precision.HIGHEST) + b_ref[...]


def _tc_finish(sums, textp2, row0, w, b2):
    return pl.pallas_call(
        _tc_body,
        out_shape=jax.ShapeDtypeStruct((_BATCH, _OUT), jnp.float32),
    )(sums, textp2, row0, w, b2)


def kernel(text, embed_weight, linear_W, linear_b):
    idx = text.astype(jnp.int32).T
    idxp = jnp.pad(idx, ((0, 0), (0, _PAD)))
    textp = idxp.reshape(2 * _BATCH, 128)
    table2 = embed_weight.reshape(_BATCH * 0 + embed_weight.shape[0] // 2,
                                  2 * _EMBED)
    sums2 = _sc_gather_sum(textp, table2)
    sums = sums2.reshape(_BATCH, _EMBED)
    row0 = table2[0:1, :_EMBED]
    out = _tc_finish(sums, idxp, row0, linear_W,
                     linear_b.reshape(1, _OUT))
    return out


# speedup vs baseline: 1.4226x; 1.4226x over previous
import jax
import jax.numpy as jnp
from jax import lax
from jax.experimental import pallas as pl
from jax.experimental.pallas import tpu as pltpu
from jax.experimental.pallas import tpu_sc as plsc

_EMBED = 64
_OUT = 2
_SEQ = 200
_BATCH = 4096
_SEQP = 256
_PAD = _SEQP - _SEQ
_G1 = 128
_G2 = _SEQ - _G1
_NC = 2
_NS = 16
_NW = _NC * _NS
_BPW = _BATCH // _NW


_VOCAB = 1000000
_CH = 4096
_NCH = (_VOCAB + _CH - 1) // _CH


def _tc_repack_body(x_ref, o_ref):
    xt = x_ref[...].T
    x3 = xt.reshape(_CH // 2, 2, _EMBED)
    evens = x3[:, 0, :]
    odds = x3[:, 1, :]
    o_ref[...] = jnp.concatenate([evens, odds], axis=1)


def _tc_repack(tableT):
    return pl.pallas_call(
        _tc_repack_body,
        grid=(_NCH,),
        in_specs=[pl.BlockSpec((_EMBED, _CH), lambda i: (0, i))],
        out_specs=pl.BlockSpec((_CH // 2, 2 * _EMBED), lambda i: (i, 0)),
        out_shape=jax.ShapeDtypeStruct((_VOCAB // 2, 2 * _EMBED), jnp.float32),
    )(tableT)


def _sc_body(textp_hbm, textq_hbm, table_hbm, sums_hbm, idx_v, idx2_v,
             rows0_v, rows1_v, out_v, sem0a, sem0b, sem1a, sem1b):
    wid = lax.axis_index("s") * _NC + lax.axis_index("c")
    base = wid * _BPW
    pltpu.sync_copy(textp_hbm.at[pl.ds(base * 2, 2 * _BPW)], idx_v)
    pltpu.sync_copy(textq_hbm.at[pl.ds(base * 2, 2 * _BPW)], idx2_v)

    def issue(b, rows_ref, sema, semb):
        pltpu.async_copy(
            table_hbm.at[idx2_v.at[2 * b]],
            rows_ref.at[pl.ds(0, _G1)], sema)
        pltpu.async_copy(
            table_hbm.at[idx2_v.at[2 * b + 1, pl.ds(0, _G2)]],
            rows_ref.at[pl.ds(_G1, _G2)], semb)

    def wait_rows(b, rows_ref, sema, semb):
        pltpu.make_async_copy(
            table_hbm.at[idx2_v.at[2 * b]],
            rows_ref.at[pl.ds(0, _G1)], sema).wait()
        pltpu.make_async_copy(
            table_hbm.at[idx2_v.at[2 * b + 1, pl.ds(0, _G2)]],
            rows_ref.at[pl.ds(_G1, _G2)], semb).wait()

    def acc_store(b, rows_ref):
        zero = jnp.zeros((16,), jnp.float32)

        def group(g, accs, row0_idx, par_row, nrows):
            par = idx_v[par_row, pl.ds(g * 16, 16)] & 1
            for j in range(nrows):
                off = par[j] * _EMBED
                s = row0_idx + g * 16 + j
                accs = tuple(
                    accs[c] + rows_ref[s, pl.ds(off + c * 16, 16)]
                    for c in range(4))
            return accs

        accs = lax.fori_loop(
            0, _G1 // 16,
            lambda g, a: group(g, a, 0, 2 * b, 16),
            (zero,) * 4)
        accs = lax.fori_loop(
            0, _G2 // 16,
            lambda g, a: group(g, a, _G1, 2 * b + 1, 16),
            accs)
        accs = group(_G2 // 16, accs, _G1, 2 * b + 1, _G2 % 16)
        half = (b & 1) * _EMBED
        for c in range(4):
            out_v[b // 2, pl.ds(half + c * 16, 16)] = accs[c]

    issue(0, rows0_v, sem0a, sem0b)

    def pairb(j, carry):
        b0 = 2 * j
        issue(b0 + 1, rows1_v, sem1a, sem1b)
        wait_rows(b0, rows0_v, sem0a, sem0b)
        acc_store(b0, rows0_v)

        @pl.when(b0 + 2 < _BPW)
        def _():
            issue(b0 + 2, rows0_v, sem0a, sem0b)
        wait_rows(b0 + 1, rows1_v, sem1a, sem1b)
        acc_store(b0 + 1, rows1_v)
        return carry

    lax.fori_loop(0, _BPW // 2, pairb, None)
    pltpu.sync_copy(out_v, sums_hbm.at[pl.ds(wid * (_BPW // 2), _BPW // 2)])


def _sc_gather_sum(textp, textq, table2):
    f = pl.kernel(
        _sc_body,
        out_type=jax.ShapeDtypeStruct((_BATCH // 2, 2 * _EMBED), jnp.float32),
        mesh=plsc.VectorSubcoreMesh(core_axis_name="c", subcore_axis_name="s"),
        scratch_types=[
            pltpu.VMEM((2 * _BPW, 128), jnp.int32),
            pltpu.VMEM((2 * _BPW, 128), jnp.int32),
            pltpu.VMEM((_SEQ, 128), jnp.float32),
            pltpu.VMEM((_SEQ, 128), jnp.float32),
            pltpu.VMEM((_BPW // 2, 2 * _EMBED), jnp.float32),
            pltpu.SemaphoreType.DMA,
            pltpu.SemaphoreType.DMA,
            pltpu.SemaphoreType.DMA,
            pltpu.SemaphoreType.DMA,
        ],
    )
    return f(textp, textq, table2)


def _tc_body(sums_ref, text_ref, row0_ref, w_ref, b_ref, out_ref):
    cnt = jnp.sum((text_ref[...] == 0).astype(jnp.float32),
                  axis=1, keepdims=True) - float(_PAD)
    pooled = (sums_ref[...] - cnt * row0_ref[...]) * (1.0 / _SEQ)
    out_ref[...] = lax.dot_general(
        pooled, w_ref[...], (((1,), (1,)), ((), ())),
        preferred_element_type=jnp.float32,
        precision=lax.Precision.HIGHEST) + b_ref[...]


def _tc_finish(sums, textp2, row0, w, b2):
    return pl.pallas_call(
        _tc_body,
        out_shape=jax.ShapeDtypeStruct((_BATCH, _OUT), jnp.float32),
    )(sums, textp2, row0, w, b2)


def kernel(text, embed_weight, linear_W, linear_b):
    idx = text.astype(jnp.int32).T
    idxp = jnp.pad(idx, ((0, 0), (0, _PAD)))
    textp = idxp.reshape(2 * _BATCH, 128)
    textq = jnp.right_shift(textp, 1)
    table2 = _tc_repack(embed_weight.T)
    sums2 = _sc_gather_sum(textp, textq, table2)
    sums = sums2.reshape(_BATCH, _EMBED)
    row0 = table2[0:1, :_EMBED]
    out = _tc_finish(sums, idxp, row0, linear_W,
                     linear_b.reshape(1, _OUT))
    return out
